# Initial kernel scaffold; baseline (speedup 1.0000x reference)
import jax
import jax.numpy as jnp
from jax.experimental import pallas as pl


def kernel(points, W, b, gamma, beta):
    raise NotImplementedError("write your pallas kernel here")


# speedup vs baseline: 1.5838x; 1.5838x over previous
import functools

import jax
import jax.numpy as jnp
from jax.experimental import pallas as pl
from jax.experimental.pallas import tpu as pltpu

PC_RANGE = (-51.2, -51.2, -3.0, 51.2, 51.2, 3.0)
VOXEL = (0.4, 0.4, 6.0)
H, WD = 256, 256
V = H * WD
FEAT = 64


CHUNK = 2048


def _h_pre(pts, g, wp, b):
    n = pts.shape[0]
    cnt = g[:, 0:1]
    mean = g[:, 1:4] / cnt
    xyz = pts[:, 0:3]
    f_cluster = xyz - mean
    x0, y0 = PC_RANGE[0], PC_RANGE[1]
    vx, vy = VOXEL[0], VOXEL[1]
    ix = jnp.floor((pts[:, 0:1] - x0) / vx)
    iy = jnp.floor((pts[:, 1:2] - y0) / vy)
    ix = jnp.clip(ix, 0.0, WD - 1.0)
    iy = jnp.clip(iy, 0.0, H - 1.0)
    cx = (ix + 0.5) * vx + x0
    cy = (iy + 0.5) * vy + y0
    f_center = jnp.concatenate([pts[:, 0:1] - cx, pts[:, 1:2] - cy], axis=1)
    aug = jnp.concatenate(
        [pts, f_cluster, f_center, jnp.zeros((n, 5), jnp.float32)], axis=1
    )
    h = jnp.dot(aug, wp, preferred_element_type=jnp.float32) + b
    return h, cnt


def _stage_c1_body(pts_ref, g_ref, wp_ref, b_ref, out_ref):
    j = pl.program_id(1)
    h, _ = _h_pre(pts_ref[0], g_ref[0], wp_ref[...], b_ref[...])
    s = jnp.sum(h, axis=0, keepdims=True)
    s2 = jnp.sum(h * h, axis=0, keepdims=True)
    part = jnp.concatenate([s, s2], axis=0)[None]

    @pl.when(j == 0)
    def _():
        out_ref[...] = part

    @pl.when(j > 0)
    def _():
        out_ref[...] += part


def _stage_c2_body(pts_ref, g_ref, wp_ref, b_ref, gamma_ref, beta_ref,
                   stats_ref, out_ref):
    h, cnt = _h_pre(pts_ref[0], g_ref[0], wp_ref[...], b_ref[...])
    n_total = 16384.0
    mu = stats_ref[0, 0:1] / n_total
    var = stats_ref[0, 1:2] / n_total - mu * mu
    hn = (h - mu) * jax.lax.rsqrt(var + 1e-3) * gamma_ref[...] + beta_ref[...]
    out_ref[0] = jnp.maximum(hn, 0.0) / cnt


def _stage_c(points, g, Wp, b, gamma, beta):
    B, N, _ = points.shape
    NC = N // CHUNK
    stats = pl.pallas_call(
        _stage_c1_body,
        grid=(B, NC),
        in_specs=[
            pl.BlockSpec((1, CHUNK, 6), lambda i, j: (i, j, 0)),
            pl.BlockSpec((1, CHUNK, 4), lambda i, j: (i, j, 0)),
            pl.BlockSpec((16, FEAT), lambda i, j: (0, 0)),
            pl.BlockSpec((1, FEAT), lambda i, j: (0, 0)),
        ],
        out_specs=pl.BlockSpec((1, 2, FEAT), lambda i, j: (i, 0, 0)),
        out_shape=jax.ShapeDtypeStruct((B, 2, FEAT), jnp.float32),
    )(points, g, Wp, b)
    return pl.pallas_call(
        _stage_c2_body,
        grid=(B, NC),
        in_specs=[
            pl.BlockSpec((1, CHUNK, 6), lambda i, j: (i, j, 0)),
            pl.BlockSpec((1, CHUNK, 4), lambda i, j: (i, j, 0)),
            pl.BlockSpec((16, FEAT), lambda i, j: (0, 0)),
            pl.BlockSpec((1, FEAT), lambda i, j: (0, 0)),
            pl.BlockSpec((1, FEAT), lambda i, j: (0, 0)),
            pl.BlockSpec((1, FEAT), lambda i, j: (0, 0)),
            pl.BlockSpec((1, 2, FEAT), lambda i, j: (i, 0, 0)),
        ],
        out_specs=pl.BlockSpec((1, CHUNK, FEAT), lambda i, j: (i, j, 0)),
        out_shape=jax.ShapeDtypeStruct((B, N, FEAT), jnp.float32),
    )(points, g, Wp, b, gamma, beta, stats)


def _stage_e_body(vf_ref, out_ref):
    out_ref[0] = vf_ref[0].T


def _stage_e(vfeat):
    B = vfeat.shape[0]
    VB = 2048
    return pl.pallas_call(
        _stage_e_body,
        grid=(B, V // VB),
        in_specs=[pl.BlockSpec((1, VB, FEAT), lambda i, j: (i, j, 0))],
        out_specs=pl.BlockSpec((1, FEAT, VB), lambda i, j: (i, 0, j)),
        out_shape=jax.ShapeDtypeStruct((B, FEAT, V), jnp.float32),
    )(vfeat)


def kernel(points, W, b, gamma, beta):
    B, N, _ = points.shape
    x0, y0 = PC_RANGE[0], PC_RANGE[1]
    vx, vy = VOXEL[0], VOXEL[1]
    ix = jnp.clip(jnp.floor((points[..., 0] - x0) / vx).astype(jnp.int32), 0, WD - 1)
    iy = jnp.clip(jnp.floor((points[..., 1] - y0) / vy).astype(jnp.int32), 0, H - 1)
    lin = iy * WD + ix

    ones = jnp.ones((B, N, 1), jnp.float32)
    rows4 = jnp.concatenate([ones, points[..., 0:3]], axis=-1)
    seg = jax.vmap(
        lambda r, l: jax.ops.segment_sum(r, l, num_segments=V)
    )(rows4, lin)
    g = jnp.take_along_axis(seg, lin[..., None], axis=1)

    Wp = jnp.concatenate([W, jnp.zeros((5, FEAT), jnp.float32)], axis=0)
    hscaled = _stage_c(points, g, Wp, b[None, :], gamma[None, :], beta[None, :])

    vfeat = jax.vmap(
        lambda h, l: jax.ops.segment_sum(h, l, num_segments=V)
    )(hscaled, lin)

    canvas = _stage_e(vfeat)
    return canvas.reshape(B, FEAT, H, WD)


# speedup vs baseline: 3.3726x; 2.1294x over previous
import functools

import jax
import jax.numpy as jnp
from jax import lax
from jax.experimental import pallas as pl
from jax.experimental.pallas import tpu as pltpu
from jax.experimental.pallas import tpu_sc as plsc

PC_RANGE = (-51.2, -51.2, -3.0, 51.2, 51.2, 3.0)
VOXEL = (0.4, 0.4, 6.0)
H, WD = 256, 256
V = H * WD
FEAT = 64
N = 16384
B = 4

NSC = 2
NSUB = 16
CHUNK = 2048
NC = N // CHUNK
NFB = 4
VL = 16


def _voxf(xs, ys):
    x0, y0 = PC_RANGE[0], PC_RANGE[1]
    vx, vy = VOXEL[0], VOXEL[1]
    ix = jnp.clip(jnp.floor((xs - x0) / vx), 0.0, WD - 1.0)
    iy = jnp.clip(jnp.floor((ys - y0) / vy), 0.0, H - 1.0)
    return ix, iy


def _a_lin_body(xp_ref, yp_ref, out_ref):
    ix, iy = _voxf(xp_ref[0], yp_ref[0])
    out_ref[0] = (iy * WD + ix).astype(jnp.int32)


def _a_lin(xr, yr):
    return pl.pallas_call(
        _a_lin_body,
        grid=(B,),
        in_specs=[
            pl.BlockSpec((1, 128, 128), lambda i: (i, 0, 0)),
            pl.BlockSpec((1, 128, 128), lambda i: (i, 0, 0)),
        ],
        out_specs=pl.BlockSpec((1, 128, 128), lambda i: (i, 0, 0)),
        out_shape=jax.ShapeDtypeStruct((B, 128, 128), jnp.int32),
    )(xr, yr)


def _zero_table(tab):
    def zi(i, _):
        tab[pl.ds(pl.multiple_of(i * VL, VL), VL)] = jnp.zeros((VL,), jnp.float32)
        return 0
    lax.fori_loop(0, V // VL, zi, 0)


def _scatter_add(tab, idx_v, val_v):
    def si(i, _):
        o = pl.multiple_of(i * VL, VL)
        iv = idx_v[pl.ds(o, VL)]
        vv = val_v[pl.ds(o, VL)]
        plsc.addupdate_scatter(tab, [iv], vv)
        return 0
    lax.fori_loop(0, N // VL, si, 0)


def _sc_b_body(vals_hbm, lin_hbm, g_hbm, idx_v, val_v, gout_v, tab):
    c = lax.axis_index("c")
    s = lax.axis_index("s")

    @pl.when(s < (B // NSC) * 4)
    def _():
        bb = c * (B // NSC) + lax.div(s, 4)
        comp = lax.rem(s, 4)
        pltpu.sync_copy(lin_hbm.at[bb], idx_v)
        pltpu.sync_copy(vals_hbm.at[bb, comp], val_v)
        _zero_table(tab)
        _scatter_add(tab, idx_v, val_v)

        def gi(i, _):
            o = pl.multiple_of(i * VL, VL)
            iv = idx_v[pl.ds(o, VL)]
            gout_v[pl.ds(o, VL)] = plsc.load_gather(tab, [iv])
            return 0

        lax.fori_loop(0, N // VL, gi, 0)
        pltpu.sync_copy(gout_v, g_hbm.at[bb, comp])


def _sc_b(vals3, lin2):
    mesh = plsc.VectorSubcoreMesh(core_axis_name="c", subcore_axis_name="s")
    f = functools.partial(
        pl.kernel,
        out_type=jax.ShapeDtypeStruct((B, 4, N), jnp.float32),
        mesh=mesh,
        compiler_params=pltpu.CompilerParams(needs_layout_passes=False),
        scratch_types=[
            pltpu.VMEM((N,), jnp.int32),
            pltpu.VMEM((N,), jnp.float32),
            pltpu.VMEM((N,), jnp.float32),
            pltpu.VMEM((V,), jnp.float32),
        ],
    )(_sc_b_body)
    return f(vals3, lin2)


def _aug_t(xp, yp, zp, f1, f2, f3, gx, gy, gz, gc):
    invc = 1.0 / gc
    mx = gx * invc
    my = gy * invc
    mz = gz * invc
    x0, y0 = PC_RANGE[0], PC_RANGE[1]
    vx, vy = VOXEL[0], VOXEL[1]
    ix, iy = _voxf(xp, yp)
    cx = (ix + 0.5) * vx + x0
    cy = (iy + 0.5) * vy + y0
    aug = jnp.concatenate([
        xp, yp, zp, f1, f2, f3,
        xp - mx, yp - my, zp - mz,
        xp - cx, yp - cy,
        jnp.zeros((5, CHUNK), jnp.float32),
    ], axis=0)
    return aug, invc


def _c1_body(xp_ref, yp_ref, zp_ref, f1_ref, f2_ref, f3_ref,
             gx_ref, gy_ref, gz_ref, gc_ref, wpt_ref, b_ref, out_ref):
    j = pl.program_id(1)
    aug, _ = _aug_t(xp_ref[0], yp_ref[0], zp_ref[0], f1_ref[0], f2_ref[0],
                    f3_ref[0], gx_ref[0], gy_ref[0], gz_ref[0], gc_ref[0])
    ht = jnp.dot(wpt_ref[...], aug, preferred_element_type=jnp.float32) + b_ref[...]
    st = jnp.sum(ht, axis=1, keepdims=True)
    s2 = jnp.sum(ht * ht, axis=1, keepdims=True)
    part = jnp.concatenate([st, s2], axis=1)[None]

    @pl.when(j == 0)
    def _():
        out_ref[...] = part

    @pl.when(j > 0)
    def _():
        out_ref[...] += part


def _c2_body(xp_ref, yp_ref, zp_ref, f1_ref, f2_ref, f3_ref,
             gx_ref, gy_ref, gz_ref, gc_ref, wpt_ref, b_ref,
             gamma_ref, beta_ref, stats_ref, out_ref):
    aug, invc = _aug_t(xp_ref[0], yp_ref[0], zp_ref[0], f1_ref[0], f2_ref[0],
                       f3_ref[0], gx_ref[0], gy_ref[0], gz_ref[0], gc_ref[0])
    ht = jnp.dot(wpt_ref[...], aug, preferred_element_type=jnp.float32) + b_ref[...]
    n_total = float(N)
    mu = stats_ref[0, :, 0:1] / n_total
    var = stats_ref[0, :, 1:2] / n_total - mu * mu
    hn = (ht - mu) * lax.rsqrt(var + 1e-3) * gamma_ref[...] + beta_ref[...]
    out_ref[0] = jnp.maximum(hn, 0.0) * invc


def _stage_c(planes, WpT, b2, gamma2, beta2):
    plane_spec = pl.BlockSpec((1, 1, CHUNK), lambda i, j: (i * NC + j, 0, 0))
    plane_specs = [plane_spec for _ in range(10)]
    w_specs = [
        pl.BlockSpec((FEAT, 16), lambda i, j: (0, 0)),
        pl.BlockSpec((FEAT, 1), lambda i, j: (0, 0)),
    ]
    stats = pl.pallas_call(
        _c1_body,
        grid=(B, NC),
        in_specs=plane_specs + w_specs,
        out_specs=pl.BlockSpec((1, FEAT, 2), lambda i, j: (i, 0, 0)),
        out_shape=jax.ShapeDtypeStruct((B, FEAT, 2), jnp.float32),
    )(*planes, WpT, b2)
    return pl.pallas_call(
        _c2_body,
        grid=(B, NC),
        in_specs=plane_specs + w_specs + [
            pl.BlockSpec((FEAT, 1), lambda i, j: (0, 0)),
            pl.BlockSpec((FEAT, 1), lambda i, j: (0, 0)),
            pl.BlockSpec((1, FEAT, 2), lambda i, j: (i, 0, 0)),
        ],
        out_specs=pl.BlockSpec((1, FEAT, CHUNK), lambda i, j: (i, 0, j)),
        out_shape=jax.ShapeDtypeStruct((B, FEAT, N), jnp.float32),
    )(*planes, WpT, b2, gamma2, beta2, stats)


def _sc_d_body(h_hbm, lin_hbm, out_hbm, idx_v, val_v, tab):
    c = lax.axis_index("c")
    s = lax.axis_index("s")
    nwave = (B // NSC) * NFB

    def wave_i(t, _):
        bb = c * (B // NSC) + lax.div(t, NFB)
        q = lax.rem(t, NFB)
        fq = q * VL + s
        pltpu.sync_copy(lin_hbm.at[bb], idx_v)
        pltpu.sync_copy(h_hbm.at[bb, fq], val_v)
        _zero_table(tab)
        _scatter_add(tab, idx_v, val_v)
        pltpu.sync_copy(tab, out_hbm.at[bb, fq])
        return 0

    lax.fori_loop(0, nwave, wave_i, 0)


def _sc_d(h3, lin2):
    mesh = plsc.VectorSubcoreMesh(core_axis_name="c", subcore_axis_name="s")
    f = functools.partial(
        pl.kernel,
        out_type=jax.ShapeDtypeStruct((B, FEAT, V), jnp.float32),
        mesh=mesh,
        compiler_params=pltpu.CompilerParams(needs_layout_passes=False),
        scratch_types=[
            pltpu.VMEM((N,), jnp.int32),
            pltpu.VMEM((N,), jnp.float32),
            pltpu.VMEM((V,), jnp.float32),
        ],
    )(_sc_d_body)
    return f(h3, lin2)


def kernel(points, W, b, gamma, beta):
    pf = [points[..., i] for i in range(6)]

    lini = _a_lin(pf[0].reshape(B, 128, 128), pf[1].reshape(B, 128, 128))
    lin2 = lini.reshape(B, N)

    vals3 = jnp.stack(
        [pf[0], pf[1], pf[2], jnp.ones((B, N), jnp.float32)], axis=1)
    g = _sc_b(vals3, lin2)
    gplanes = [g[:, i].reshape(B * NC, 1, CHUNK) for i in range(4)]

    planes = [p.reshape(B * NC, 1, CHUNK) for p in pf] + gplanes
    WpT = jnp.concatenate([W, jnp.zeros((5, FEAT), jnp.float32)], axis=0).T
    h = _stage_c(planes, WpT, b[:, None], gamma[:, None], beta[:, None])

    canvas = _sc_d(h, lin2)
    return canvas.reshape(B, FEAT, H, WD)


# speedup vs baseline: 4.3914x; 1.3021x over previous
import functools

import jax
import jax.numpy as jnp
from jax import lax
from jax.experimental import pallas as pl
from jax.experimental.pallas import tpu as pltpu
from jax.experimental.pallas import tpu_sc as plsc

PC_RANGE = (-51.2, -51.2, -3.0, 51.2, 51.2, 3.0)
VOXEL = (0.4, 0.4, 6.0)
H, WD = 256, 256
V = H * WD
FEAT = 64
N = 16384
B = 4

NSC = 2
NSUB = 16
CHUNK = 2048
NC = N // CHUNK
NFB = 4
VL = 16


def _voxf(xs, ys):
    x0, y0 = PC_RANGE[0], PC_RANGE[1]
    vx, vy = VOXEL[0], VOXEL[1]
    ix = jnp.clip(jnp.floor((xs - x0) / vx), 0.0, WD - 1.0)
    iy = jnp.clip(jnp.floor((ys - y0) / vy), 0.0, H - 1.0)
    return ix, iy


def _a_lin_body(xp_ref, yp_ref, out_ref):
    ix, iy = _voxf(xp_ref[0], yp_ref[0])
    out_ref[0] = (iy * WD + ix).astype(jnp.int32)


def _a_lin(xr, yr):
    return pl.pallas_call(
        _a_lin_body,
        grid=(B,),
        in_specs=[
            pl.BlockSpec((1, 128, 128), lambda i: (i, 0, 0)),
            pl.BlockSpec((1, 128, 128), lambda i: (i, 0, 0)),
        ],
        out_specs=pl.BlockSpec((1, 128, 128), lambda i: (i, 0, 0)),
        out_shape=jax.ShapeDtypeStruct((B, 128, 128), jnp.int32),
    )(xr, yr)


def _zero_table(tab):
    def zi(i, _):
        tab[pl.ds(pl.multiple_of(i * VL, VL), VL)] = jnp.zeros((VL,), jnp.float32)
        return 0
    lax.fori_loop(0, V // VL, zi, 0)


def _scatter_add(tab, idx_v, val_v):
    def si(i, _):
        o = pl.multiple_of(i * VL, VL)
        iv = idx_v[pl.ds(o, VL)]
        vv = val_v[pl.ds(o, VL)]
        plsc.addupdate_scatter(tab, [iv], vv)
        return 0
    lax.fori_loop(0, N // VL, si, 0)


def _scatter_zero(tab, idx_v):
    zv = jnp.zeros((VL,), jnp.float32)

    def si(i, _):
        o = pl.multiple_of(i * VL, VL)
        iv = idx_v[pl.ds(o, VL)]
        plsc.store_scatter(tab, [iv], zv)
        return 0
    lax.fori_loop(0, N // VL, si, 0)


def _sc_b_body(vals_hbm, lin_hbm, g_hbm, idx_v, val_v, gout_v, tab):
    c = lax.axis_index("c")
    s = lax.axis_index("s")

    @pl.when(s < (B // NSC) * 4)
    def _():
        bb = c * (B // NSC) + lax.div(s, 4)
        comp = lax.rem(s, 4)
        pltpu.sync_copy(lin_hbm.at[bb], idx_v)
        pltpu.sync_copy(vals_hbm.at[bb, comp], val_v)
        _zero_table(tab)
        _scatter_add(tab, idx_v, val_v)

        def gi(i, _):
            o = pl.multiple_of(i * VL, VL)
            iv = idx_v[pl.ds(o, VL)]
            gout_v[pl.ds(o, VL)] = plsc.load_gather(tab, [iv])
            return 0

        lax.fori_loop(0, N // VL, gi, 0)
        pltpu.sync_copy(gout_v, g_hbm.at[bb, comp])


def _sc_b(vals3, lin2):
    mesh = plsc.VectorSubcoreMesh(core_axis_name="c", subcore_axis_name="s")
    f = functools.partial(
        pl.kernel,
        out_type=jax.ShapeDtypeStruct((B, 4, N), jnp.float32),
        mesh=mesh,
        compiler_params=pltpu.CompilerParams(needs_layout_passes=False),
        scratch_types=[
            pltpu.VMEM((N,), jnp.int32),
            pltpu.VMEM((N,), jnp.float32),
            pltpu.VMEM((N,), jnp.float32),
            pltpu.VMEM((V,), jnp.float32),
        ],
    )(_sc_b_body)
    return f(vals3, lin2)


def _aug_t(xp, yp, zp, f1, f2, f3, gx, gy, gz, gc):
    invc = 1.0 / gc
    mx = gx * invc
    my = gy * invc
    mz = gz * invc
    x0, y0 = PC_RANGE[0], PC_RANGE[1]
    vx, vy = VOXEL[0], VOXEL[1]
    ix, iy = _voxf(xp, yp)
    cx = (ix + 0.5) * vx + x0
    cy = (iy + 0.5) * vy + y0
    aug = jnp.concatenate([
        xp, yp, zp, f1, f2, f3,
        xp - mx, yp - my, zp - mz,
        xp - cx, yp - cy,
        jnp.zeros((5, CHUNK), jnp.float32),
    ], axis=0)
    return aug, invc


def _c1_body(xp_ref, yp_ref, zp_ref, f1_ref, f2_ref, f3_ref,
             gx_ref, gy_ref, gz_ref, gc_ref, wpt_ref, b_ref, out_ref):
    j = pl.program_id(1)
    aug, _ = _aug_t(xp_ref[0], yp_ref[0], zp_ref[0], f1_ref[0], f2_ref[0],
                    f3_ref[0], gx_ref[0], gy_ref[0], gz_ref[0], gc_ref[0])
    ht = jnp.dot(wpt_ref[...], aug, preferred_element_type=jnp.float32) + b_ref[...]
    st = jnp.sum(ht, axis=1, keepdims=True)
    s2 = jnp.sum(ht * ht, axis=1, keepdims=True)
    part = jnp.concatenate([st, s2], axis=1)[None]

    @pl.when(j == 0)
    def _():
        out_ref[...] = part

    @pl.when(j > 0)
    def _():
        out_ref[...] += part


def _c2_body(xp_ref, yp_ref, zp_ref, f1_ref, f2_ref, f3_ref,
             gx_ref, gy_ref, gz_ref, gc_ref, wpt_ref, b_ref,
             gamma_ref, beta_ref, stats_ref, out_ref):
    aug, invc = _aug_t(xp_ref[0], yp_ref[0], zp_ref[0], f1_ref[0], f2_ref[0],
                       f3_ref[0], gx_ref[0], gy_ref[0], gz_ref[0], gc_ref[0])
    ht = jnp.dot(wpt_ref[...], aug, preferred_element_type=jnp.float32) + b_ref[...]
    n_total = float(N)
    mu = stats_ref[0, :, 0:1] / n_total
    var = stats_ref[0, :, 1:2] / n_total - mu * mu
    hn = (ht - mu) * lax.rsqrt(var + 1e-3) * gamma_ref[...] + beta_ref[...]
    out_ref[0] = jnp.maximum(hn, 0.0) * invc


def _stage_c(planes, WpT, b2, gamma2, beta2):
    plane_spec = pl.BlockSpec((1, 1, CHUNK), lambda i, j: (i * NC + j, 0, 0))
    plane_specs = [plane_spec for _ in range(10)]
    w_specs = [
        pl.BlockSpec((FEAT, 16), lambda i, j: (0, 0)),
        pl.BlockSpec((FEAT, 1), lambda i, j: (0, 0)),
    ]
    stats = pl.pallas_call(
        _c1_body,
        grid=(B, NC),
        in_specs=plane_specs + w_specs,
        out_specs=pl.BlockSpec((1, FEAT, 2), lambda i, j: (i, 0, 0)),
        out_shape=jax.ShapeDtypeStruct((B, FEAT, 2), jnp.float32),
    )(*planes, WpT, b2)
    return pl.pallas_call(
        _c2_body,
        grid=(B, NC),
        in_specs=plane_specs + w_specs + [
            pl.BlockSpec((FEAT, 1), lambda i, j: (0, 0)),
            pl.BlockSpec((FEAT, 1), lambda i, j: (0, 0)),
            pl.BlockSpec((1, FEAT, 2), lambda i, j: (i, 0, 0)),
        ],
        out_specs=pl.BlockSpec((1, FEAT, CHUNK), lambda i, j: (i, 0, j)),
        out_shape=jax.ShapeDtypeStruct((B, FEAT, N), jnp.float32),
    )(*planes, WpT, b2, gamma2, beta2, stats)


def _sc_d_body(h_hbm, lin_hbm, out_hbm, idx_v, val_v, tab):
    c = lax.axis_index("c")
    s = lax.axis_index("s")
    nwave = (B // NSC) * NFB

    _zero_table(tab)

    def wave_i(t, _):
        bb = c * (B // NSC) + lax.div(t, NFB)
        q = lax.rem(t, NFB)
        fq = q * VL + s

        @pl.when(q == 0)
        def _():
            pltpu.sync_copy(lin_hbm.at[bb], idx_v)

        pltpu.sync_copy(h_hbm.at[bb, fq], val_v)
        _scatter_add(tab, idx_v, val_v)
        pltpu.sync_copy(tab, out_hbm.at[bb, fq])
        _scatter_zero(tab, idx_v)
        return 0

    lax.fori_loop(0, nwave, wave_i, 0)


def _sc_d(h3, lin2):
    mesh = plsc.VectorSubcoreMesh(core_axis_name="c", subcore_axis_name="s")
    f = functools.partial(
        pl.kernel,
        out_type=jax.ShapeDtypeStruct((B, FEAT, V), jnp.float32),
        mesh=mesh,
        compiler_params=pltpu.CompilerParams(needs_layout_passes=False),
        scratch_types=[
            pltpu.VMEM((N,), jnp.int32),
            pltpu.VMEM((N,), jnp.float32),
            pltpu.VMEM((V,), jnp.float32),
        ],
    )(_sc_d_body)
    return f(h3, lin2)


def kernel(points, W, b, gamma, beta):
    pf = [points[..., i] for i in range(6)]

    lini = _a_lin(pf[0].reshape(B, 128, 128), pf[1].reshape(B, 128, 128))
    lin2 = lini.reshape(B, N)

    vals3 = jnp.stack(
        [pf[0], pf[1], pf[2], jnp.ones((B, N), jnp.float32)], axis=1)
    g = _sc_b(vals3, lin2)
    gplanes = [g[:, i].reshape(B * NC, 1, CHUNK) for i in range(4)]

    planes = [p.reshape(B * NC, 1, CHUNK) for p in pf] + gplanes
    WpT = jnp.concatenate([W, jnp.zeros((5, FEAT), jnp.float32)], axis=0).T
    h = _stage_c(planes, WpT, b[:, None], gamma[:, None], beta[:, None])

    canvas = _sc_d(h, lin2)
    return canvas.reshape(B, FEAT, H, WD)


# speedup vs baseline: 4.9796x; 1.1339x over previous
import functools

import jax
import jax.numpy as jnp
from jax import lax
from jax.experimental import pallas as pl
from jax.experimental.pallas import tpu as pltpu
from jax.experimental.pallas import tpu_sc as plsc

PC_RANGE = (-51.2, -51.2, -3.0, 51.2, 51.2, 3.0)
VOXEL = (0.4, 0.4, 6.0)
H, WD = 256, 256
V = H * WD
FEAT = 64
N = 16384
B = 4

NSC = 2
NSUB = 16
CHUNK = 2048
NC = N // CHUNK
NFB = 4
VL = 16


def _voxf(xs, ys):
    x0, y0 = PC_RANGE[0], PC_RANGE[1]
    vx, vy = VOXEL[0], VOXEL[1]
    ix = jnp.clip(jnp.floor((xs - x0) / vx), 0.0, WD - 1.0)
    iy = jnp.clip(jnp.floor((ys - y0) / vy), 0.0, H - 1.0)
    return ix, iy


def _a_lin_body(xp_ref, yp_ref, out_ref):
    ix, iy = _voxf(xp_ref[0], yp_ref[0])
    out_ref[0] = (iy * WD + ix).astype(jnp.int32)


def _a_lin(xr, yr):
    return pl.pallas_call(
        _a_lin_body,
        grid=(B,),
        in_specs=[
            pl.BlockSpec((1, 128, 128), lambda i: (i, 0, 0)),
            pl.BlockSpec((1, 128, 128), lambda i: (i, 0, 0)),
        ],
        out_specs=pl.BlockSpec((1, 128, 128), lambda i: (i, 0, 0)),
        out_shape=jax.ShapeDtypeStruct((B, 128, 128), jnp.int32),
    )(xr, yr)


UNROLL = 4


def _zero_table(tab):
    zv = jnp.zeros((VL,), jnp.float32)

    def zi(i, _):
        for u in range(UNROLL):
            o = pl.multiple_of(i * (VL * UNROLL) + u * VL, VL)
            tab[pl.ds(o, VL)] = zv
        return 0
    lax.fori_loop(0, V // (VL * UNROLL), zi, 0)


def _scatter_add(tab, idx_v, val_v):
    def si(i, _):
        for u in range(UNROLL):
            o = pl.multiple_of(i * (VL * UNROLL) + u * VL, VL)
            iv = idx_v[pl.ds(o, VL)]
            vv = val_v[pl.ds(o, VL)]
            plsc.addupdate_scatter(tab, [iv], vv)
        return 0
    lax.fori_loop(0, N // (VL * UNROLL), si, 0)


def _scatter_zero(tab, idx_v):
    zv = jnp.zeros((VL,), jnp.float32)

    def si(i, _):
        for u in range(UNROLL):
            o = pl.multiple_of(i * (VL * UNROLL) + u * VL, VL)
            iv = idx_v[pl.ds(o, VL)]
            plsc.store_scatter(tab, [iv], zv)
        return 0
    lax.fori_loop(0, N // (VL * UNROLL), si, 0)


def _sc_b_body(vals_hbm, lin_hbm, g_hbm, idx_v, val_v, gout_v, tab):
    c = lax.axis_index("c")
    s = lax.axis_index("s")

    @pl.when(s < (B // NSC) * 4)
    def _():
        bb = c * (B // NSC) + lax.div(s, 4)
        comp = lax.rem(s, 4)
        pltpu.sync_copy(lin_hbm.at[bb], idx_v)
        pltpu.sync_copy(vals_hbm.at[bb, comp], val_v)
        _zero_table(tab)
        _scatter_add(tab, idx_v, val_v)

        def gi(i, _):
            for u in range(UNROLL):
                o = pl.multiple_of(i * (VL * UNROLL) + u * VL, VL)
                iv = idx_v[pl.ds(o, VL)]
                gout_v[pl.ds(o, VL)] = plsc.load_gather(tab, [iv])
            return 0

        lax.fori_loop(0, N // (VL * UNROLL), gi, 0)
        pltpu.sync_copy(gout_v, g_hbm.at[bb, comp])


def _sc_b(vals3, lin2):
    mesh = plsc.VectorSubcoreMesh(core_axis_name="c", subcore_axis_name="s")
    f = functools.partial(
        pl.kernel,
        out_type=jax.ShapeDtypeStruct((B, 4, N), jnp.float32),
        mesh=mesh,
        compiler_params=pltpu.CompilerParams(needs_layout_passes=False),
        scratch_types=[
            pltpu.VMEM((N,), jnp.int32),
            pltpu.VMEM((N,), jnp.float32),
            pltpu.VMEM((N,), jnp.float32),
            pltpu.VMEM((V,), jnp.float32),
        ],
    )(_sc_b_body)
    return f(vals3, lin2)


def _aug_t(xp, yp, zp, f1, f2, f3, gx, gy, gz, gc):
    invc = 1.0 / gc
    mx = gx * invc
    my = gy * invc
    mz = gz * invc
    x0, y0 = PC_RANGE[0], PC_RANGE[1]
    vx, vy = VOXEL[0], VOXEL[1]
    ix, iy = _voxf(xp, yp)
    cx = (ix + 0.5) * vx + x0
    cy = (iy + 0.5) * vy + y0
    aug = jnp.concatenate([
        xp, yp, zp, f1, f2, f3,
        xp - mx, yp - my, zp - mz,
        xp - cx, yp - cy,
        jnp.zeros((5, CHUNK), jnp.float32),
    ], axis=0)
    return aug, invc


def _c1_body(xp_ref, yp_ref, zp_ref, f1_ref, f2_ref, f3_ref,
             gx_ref, gy_ref, gz_ref, gc_ref, wpt_ref, b_ref, out_ref):
    j = pl.program_id(1)
    aug, _ = _aug_t(xp_ref[0], yp_ref[0], zp_ref[0], f1_ref[0], f2_ref[0],
                    f3_ref[0], gx_ref[0], gy_ref[0], gz_ref[0], gc_ref[0])
    ht = jnp.dot(wpt_ref[...], aug, preferred_element_type=jnp.float32) + b_ref[...]
    st = jnp.sum(ht, axis=1, keepdims=True)
    s2 = jnp.sum(ht * ht, axis=1, keepdims=True)
    part = jnp.concatenate([st, s2], axis=1)[None]

    @pl.when(j == 0)
    def _():
        out_ref[...] = part

    @pl.when(j > 0)
    def _():
        out_ref[...] += part


def _c2_body(xp_ref, yp_ref, zp_ref, f1_ref, f2_ref, f3_ref,
             gx_ref, gy_ref, gz_ref, gc_ref, wpt_ref, b_ref,
             gamma_ref, beta_ref, stats_ref, out_ref):
    aug, invc = _aug_t(xp_ref[0], yp_ref[0], zp_ref[0], f1_ref[0], f2_ref[0],
                       f3_ref[0], gx_ref[0], gy_ref[0], gz_ref[0], gc_ref[0])
    ht = jnp.dot(wpt_ref[...], aug, preferred_element_type=jnp.float32) + b_ref[...]
    n_total = float(N)
    mu = stats_ref[0, :, 0:1] / n_total
    var = stats_ref[0, :, 1:2] / n_total - mu * mu
    hn = (ht - mu) * lax.rsqrt(var + 1e-3) * gamma_ref[...] + beta_ref[...]
    out_ref[0] = jnp.maximum(hn, 0.0) * invc


def _stage_c(planes, WpT, b2, gamma2, beta2):
    plane_spec = pl.BlockSpec((1, 1, CHUNK), lambda i, j: (i * NC + j, 0, 0))
    plane_specs = [plane_spec for _ in range(10)]
    w_specs = [
        pl.BlockSpec((FEAT, 16), lambda i, j: (0, 0)),
        pl.BlockSpec((FEAT, 1), lambda i, j: (0, 0)),
    ]
    stats = pl.pallas_call(
        _c1_body,
        grid=(B, NC),
        in_specs=plane_specs + w_specs,
        out_specs=pl.BlockSpec((1, FEAT, 2), lambda i, j: (i, 0, 0)),
        out_shape=jax.ShapeDtypeStruct((B, FEAT, 2), jnp.float32),
    )(*planes, WpT, b2)
    return pl.pallas_call(
        _c2_body,
        grid=(B, NC),
        in_specs=plane_specs + w_specs + [
            pl.BlockSpec((FEAT, 1), lambda i, j: (0, 0)),
            pl.BlockSpec((FEAT, 1), lambda i, j: (0, 0)),
            pl.BlockSpec((1, FEAT, 2), lambda i, j: (i, 0, 0)),
        ],
        out_specs=pl.BlockSpec((1, FEAT, CHUNK), lambda i, j: (i, 0, j)),
        out_shape=jax.ShapeDtypeStruct((B, FEAT, N), jnp.float32),
    )(*planes, WpT, b2, gamma2, beta2, stats)


def _sc_d_body(h_hbm, lin_hbm, out_hbm, idx_v, val_v, tab):
    c = lax.axis_index("c")
    s = lax.axis_index("s")
    nwave = (B // NSC) * NFB

    _zero_table(tab)

    def wave_i(t, _):
        bb = c * (B // NSC) + lax.div(t, NFB)
        q = lax.rem(t, NFB)
        fq = q * VL + s

        @pl.when(q == 0)
        def _():
            pltpu.sync_copy(lin_hbm.at[bb], idx_v)

        pltpu.sync_copy(h_hbm.at[bb, fq], val_v)
        _scatter_add(tab, idx_v, val_v)
        pltpu.sync_copy(tab, out_hbm.at[bb, fq])
        _scatter_zero(tab, idx_v)
        return 0

    lax.fori_loop(0, nwave, wave_i, 0)


def _sc_d(h3, lin2):
    mesh = plsc.VectorSubcoreMesh(core_axis_name="c", subcore_axis_name="s")
    f = functools.partial(
        pl.kernel,
        out_type=jax.ShapeDtypeStruct((B, FEAT, V), jnp.float32),
        mesh=mesh,
        compiler_params=pltpu.CompilerParams(needs_layout_passes=False),
        scratch_types=[
            pltpu.VMEM((N,), jnp.int32),
            pltpu.VMEM((N,), jnp.float32),
            pltpu.VMEM((V,), jnp.float32),
        ],
    )(_sc_d_body)
    return f(h3, lin2)


def kernel(points, W, b, gamma, beta):
    pf = [points[..., i] for i in range(6)]

    lini = _a_lin(pf[0].reshape(B, 128, 128), pf[1].reshape(B, 128, 128))
    lin2 = lini.reshape(B, N)

    vals3 = jnp.stack(
        [pf[0], pf[1], pf[2], jnp.ones((B, N), jnp.float32)], axis=1)
    g = _sc_b(vals3, lin2)
    gplanes = [g[:, i].reshape(B * NC, 1, CHUNK) for i in range(4)]

    planes = [p.reshape(B * NC, 1, CHUNK) for p in pf] + gplanes
    WpT = jnp.concatenate([W, jnp.zeros((5, FEAT), jnp.float32)], axis=0).T
    h = _stage_c(planes, WpT, b[:, None], gamma[:, None], beta[:, None])

    canvas = _sc_d(h, lin2)
    return canvas.reshape(B, FEAT, H, WD)


# speedup vs baseline: 5.1016x; 1.0245x over previous
import functools

import jax
import jax.numpy as jnp
from jax import lax
from jax.experimental import pallas as pl
from jax.experimental.pallas import tpu as pltpu
from jax.experimental.pallas import tpu_sc as plsc

PC_RANGE = (-51.2, -51.2, -3.0, 51.2, 51.2, 3.0)
VOXEL = (0.4, 0.4, 6.0)
H, WD = 256, 256
V = H * WD
FEAT = 64
N = 16384
B = 4

NSC = 2
NSUB = 16
CHUNK = 2048
NC = N // CHUNK
NFB = 4
VL = 16


def _voxf(xs, ys):
    x0, y0 = PC_RANGE[0], PC_RANGE[1]
    vx, vy = VOXEL[0], VOXEL[1]
    ix = jnp.clip(jnp.floor((xs - x0) / vx), 0.0, WD - 1.0)
    iy = jnp.clip(jnp.floor((ys - y0) / vy), 0.0, H - 1.0)
    return ix, iy


def _a_lin_body(xp_ref, yp_ref, out_ref):
    ix, iy = _voxf(xp_ref[0], yp_ref[0])
    out_ref[0] = (iy * WD + ix).astype(jnp.int32)


def _a_lin(xr, yr):
    return pl.pallas_call(
        _a_lin_body,
        grid=(B,),
        in_specs=[
            pl.BlockSpec((1, 128, 128), lambda i: (i, 0, 0)),
            pl.BlockSpec((1, 128, 128), lambda i: (i, 0, 0)),
        ],
        out_specs=pl.BlockSpec((1, 128, 128), lambda i: (i, 0, 0)),
        out_shape=jax.ShapeDtypeStruct((B, 128, 128), jnp.int32),
    )(xr, yr)


UNROLL = 8


def _zero_table(tab):
    zv = jnp.zeros((VL,), jnp.float32)

    def zi(i, _):
        for u in range(UNROLL):
            o = pl.multiple_of(i * (VL * UNROLL) + u * VL, VL)
            tab[pl.ds(o, VL)] = zv
        return 0
    lax.fori_loop(0, V // (VL * UNROLL), zi, 0)


def _scatter_add(tab, idx_v, val_v):
    def si(i, _):
        for u in range(UNROLL):
            o = pl.multiple_of(i * (VL * UNROLL) + u * VL, VL)
            iv = idx_v[pl.ds(o, VL)]
            vv = val_v[pl.ds(o, VL)]
            plsc.addupdate_scatter(tab, [iv], vv)
        return 0
    lax.fori_loop(0, N // (VL * UNROLL), si, 0)


def _scatter_zero(tab, idx_v):
    zv = jnp.zeros((VL,), jnp.float32)

    def si(i, _):
        for u in range(UNROLL):
            o = pl.multiple_of(i * (VL * UNROLL) + u * VL, VL)
            iv = idx_v[pl.ds(o, VL)]
            plsc.store_scatter(tab, [iv], zv)
        return 0
    lax.fori_loop(0, N // (VL * UNROLL), si, 0)


def _sc_b_body(vals_hbm, lin_hbm, g_hbm, idx_v, val_v, gout_v, tab):
    c = lax.axis_index("c")
    s = lax.axis_index("s")

    @pl.when(s < (B // NSC) * 4)
    def _():
        bb = c * (B // NSC) + lax.div(s, 4)
        comp = lax.rem(s, 4)
        pltpu.sync_copy(lin_hbm.at[bb], idx_v)
        pltpu.sync_copy(vals_hbm.at[bb, comp], val_v)
        _zero_table(tab)
        _scatter_add(tab, idx_v, val_v)

        def gi(i, _):
            for u in range(UNROLL):
                o = pl.multiple_of(i * (VL * UNROLL) + u * VL, VL)
                iv = idx_v[pl.ds(o, VL)]
                gout_v[pl.ds(o, VL)] = plsc.load_gather(tab, [iv])
            return 0

        lax.fori_loop(0, N // (VL * UNROLL), gi, 0)
        pltpu.sync_copy(gout_v, g_hbm.at[bb, comp])


def _sc_b(vals3, lin2):
    mesh = plsc.VectorSubcoreMesh(core_axis_name="c", subcore_axis_name="s")
    f = functools.partial(
        pl.kernel,
        out_type=jax.ShapeDtypeStruct((B, 4, N), jnp.float32),
        mesh=mesh,
        compiler_params=pltpu.CompilerParams(needs_layout_passes=False),
        scratch_types=[
            pltpu.VMEM((N,), jnp.int32),
            pltpu.VMEM((N,), jnp.float32),
            pltpu.VMEM((N,), jnp.float32),
            pltpu.VMEM((V,), jnp.float32),
        ],
    )(_sc_b_body)
    return f(vals3, lin2)


def _aug_t(xp, yp, zp, f1, f2, f3, gx, gy, gz, gc):
    invc = 1.0 / gc
    mx = gx * invc
    my = gy * invc
    mz = gz * invc
    x0, y0 = PC_RANGE[0], PC_RANGE[1]
    vx, vy = VOXEL[0], VOXEL[1]
    ix, iy = _voxf(xp, yp)
    cx = (ix + 0.5) * vx + x0
    cy = (iy + 0.5) * vy + y0
    aug = jnp.concatenate([
        xp, yp, zp, f1, f2, f3,
        xp - mx, yp - my, zp - mz,
        xp - cx, yp - cy,
        jnp.zeros((5, CHUNK), jnp.float32),
    ], axis=0)
    return aug, invc


def _c1_body(xp_ref, yp_ref, zp_ref, f1_ref, f2_ref, f3_ref,
             gx_ref, gy_ref, gz_ref, gc_ref, wpt_ref, b_ref, out_ref):
    j = pl.program_id(1)
    aug, _ = _aug_t(xp_ref[0], yp_ref[0], zp_ref[0], f1_ref[0], f2_ref[0],
                    f3_ref[0], gx_ref[0], gy_ref[0], gz_ref[0], gc_ref[0])
    ht = jnp.dot(wpt_ref[...], aug, preferred_element_type=jnp.float32) + b_ref[...]
    st = jnp.sum(ht, axis=1, keepdims=True)
    s2 = jnp.sum(ht * ht, axis=1, keepdims=True)
    part = jnp.concatenate([st, s2], axis=1)[None]

    @pl.when(j == 0)
    def _():
        out_ref[...] = part

    @pl.when(j > 0)
    def _():
        out_ref[...] += part


def _c2_body(xp_ref, yp_ref, zp_ref, f1_ref, f2_ref, f3_ref,
             gx_ref, gy_ref, gz_ref, gc_ref, wpt_ref, b_ref,
             gamma_ref, beta_ref, stats_ref, out_ref):
    aug, invc = _aug_t(xp_ref[0], yp_ref[0], zp_ref[0], f1_ref[0], f2_ref[0],
                       f3_ref[0], gx_ref[0], gy_ref[0], gz_ref[0], gc_ref[0])
    ht = jnp.dot(wpt_ref[...], aug, preferred_element_type=jnp.float32) + b_ref[...]
    n_total = float(N)
    mu = stats_ref[0, :, 0:1] / n_total
    var = stats_ref[0, :, 1:2] / n_total - mu * mu
    hn = (ht - mu) * lax.rsqrt(var + 1e-3) * gamma_ref[...] + beta_ref[...]
    out_ref[0] = jnp.maximum(hn, 0.0) * invc


def _stage_c(planes, WpT, b2, gamma2, beta2):
    plane_spec = pl.BlockSpec((1, 1, CHUNK), lambda i, j: (i * NC + j, 0, 0))
    plane_specs = [plane_spec for _ in range(10)]
    w_specs = [
        pl.BlockSpec((FEAT, 16), lambda i, j: (0, 0)),
        pl.BlockSpec((FEAT, 1), lambda i, j: (0, 0)),
    ]
    stats = pl.pallas_call(
        _c1_body,
        grid=(B, NC),
        in_specs=plane_specs + w_specs,
        out_specs=pl.BlockSpec((1, FEAT, 2), lambda i, j: (i, 0, 0)),
        out_shape=jax.ShapeDtypeStruct((B, FEAT, 2), jnp.float32),
    )(*planes, WpT, b2)
    return pl.pallas_call(
        _c2_body,
        grid=(B, NC),
        in_specs=plane_specs + w_specs + [
            pl.BlockSpec((FEAT, 1), lambda i, j: (0, 0)),
            pl.BlockSpec((FEAT, 1), lambda i, j: (0, 0)),
            pl.BlockSpec((1, FEAT, 2), lambda i, j: (i, 0, 0)),
        ],
        out_specs=pl.BlockSpec((1, FEAT, CHUNK), lambda i, j: (i, 0, j)),
        out_shape=jax.ShapeDtypeStruct((B, FEAT, N), jnp.float32),
    )(*planes, WpT, b2, gamma2, beta2, stats)


def _sc_d_body(h_hbm, lin_hbm, out_hbm, idx_v, val_v, tab):
    c = lax.axis_index("c")
    s = lax.axis_index("s")
    nwave = (B // NSC) * NFB

    _zero_table(tab)

    def wave_i(t, _):
        bb = c * (B // NSC) + lax.div(t, NFB)
        q = lax.rem(t, NFB)
        fq = q * VL + s

        @pl.when(q == 0)
        def _():
            pltpu.sync_copy(lin_hbm.at[bb], idx_v)

        pltpu.sync_copy(h_hbm.at[bb, fq], val_v)
        _scatter_add(tab, idx_v, val_v)
        pltpu.sync_copy(tab, out_hbm.at[bb, fq])
        _scatter_zero(tab, idx_v)
        return 0

    lax.fori_loop(0, nwave, wave_i, 0)


def _sc_d(h3, lin2):
    mesh = plsc.VectorSubcoreMesh(core_axis_name="c", subcore_axis_name="s")
    f = functools.partial(
        pl.kernel,
        out_type=jax.ShapeDtypeStruct((B, FEAT, V), jnp.float32),
        mesh=mesh,
        compiler_params=pltpu.CompilerParams(needs_layout_passes=False),
        scratch_types=[
            pltpu.VMEM((N,), jnp.int32),
            pltpu.VMEM((N,), jnp.float32),
            pltpu.VMEM((V,), jnp.float32),
        ],
    )(_sc_d_body)
    return f(h3, lin2)


def kernel(points, W, b, gamma, beta):
    pf = [points[..., i] for i in range(6)]

    lini = _a_lin(pf[0].reshape(B, 128, 128), pf[1].reshape(B, 128, 128))
    lin2 = lini.reshape(B, N)

    vals3 = jnp.stack(
        [pf[0], pf[1], pf[2], jnp.ones((B, N), jnp.float32)], axis=1)
    g = _sc_b(vals3, lin2)
    gplanes = [g[:, i].reshape(B * NC, 1, CHUNK) for i in range(4)]

    planes = [p.reshape(B * NC, 1, CHUNK) for p in pf] + gplanes
    WpT = jnp.concatenate([W, jnp.zeros((5, FEAT), jnp.float32)], axis=0).T
    h = _stage_c(planes, WpT, b[:, None], gamma[:, None], beta[:, None])

    canvas = _sc_d(h, lin2)
    return canvas.reshape(B, FEAT, H, WD)
